# baseline (speedup 1.0000x reference)
import jax
import jax.numpy as jnp
from jax import lax
from jax.experimental import pallas as pl
from jax.experimental.pallas import tpu as pltpu
from jax.experimental.pallas import tpu_sc as plsc

D = 64
DP = 128
NC = 2
NS = 16
NW = NC * NS
B_TOTAL = 4096 * 200
B_PER_W = B_TOTAL // NW
GRP = 128
CHUNK = 128
NGATHER = CHUNK // GRP
NBUF = 4
N_CHUNKS = B_PER_W // CHUNK


def _body(idx_hbm, table_hbm, out_hbm, idx_v, rows_v, gsems, wsems):
    wid = lax.axis_index("s") * NC + lax.axis_index("c")
    base = wid * B_PER_W
    pltpu.sync_copy(idx_hbm.at[pl.ds(base, B_PER_W)], idx_v)

    def issue_gathers(g, b):
        off = g * CHUNK
        for j in range(NGATHER):
            pltpu.async_copy(
                table_hbm.at[idx_v.at[pl.ds(off + j * GRP, GRP)]],
                rows_v.at[b, pl.ds(j * GRP, GRP)],
                gsems[b],
            )

    def wait_gathers(b):
        for j in range(NGATHER):
            pltpu.make_async_copy(
                table_hbm.at[idx_v.at[pl.ds(j * GRP, GRP)]],
                rows_v.at[b, pl.ds(j * GRP, GRP)],
                gsems[b],
            ).wait()

    def wait_writeback(b):
        pltpu.make_async_copy(
            rows_v.at[b],
            out_hbm.at[pl.ds(base, CHUNK)],
            wsems[b],
        ).wait()

    issue_gathers(0, 0)
    issue_gathers(1, 1)

    @pl.loop(0, N_CHUNKS // NBUF)
    def _step(step):
        for b in range(NBUF):
            g = step * NBUF + b
            bn = (b + 2) % NBUF
            gn = g + 2

            @pl.when(gn < N_CHUNKS)
            def _():
                @pl.when(gn >= NBUF)
                def _():
                    wait_writeback(bn)
                issue_gathers(gn, bn)

            wait_gathers(b)
            pltpu.async_copy(
                rows_v.at[b],
                out_hbm.at[pl.ds(base + g * CHUNK, CHUNK)],
                wsems[b],
            )

    for b in range(NBUF):
        wait_writeback(b)


def _make_kernel():
    return pl.kernel(
        _body,
        out_type=jax.ShapeDtypeStruct((B_TOTAL, DP), jnp.float32),
        mesh=plsc.VectorSubcoreMesh(core_axis_name="c", subcore_axis_name="s"),
        scratch_types=[
            pltpu.VMEM((B_PER_W,), jnp.int32),
            pltpu.VMEM((NBUF, CHUNK, DP), jnp.float32),
            [pltpu.SemaphoreType.DMA] * NBUF,
            [pltpu.SemaphoreType.DMA] * NBUF,
        ],
        compiler_params=pltpu.CompilerParams(use_tc_tiling_on_sc=True),
    )


def kernel(text, embedding_weight):
    table_pad = jnp.pad(embedding_weight, ((0, 0), (0, DP - D)))
    flat = text.reshape(-1)
    out = _make_kernel()(flat, table_pad)
    return out[:, :D].reshape(text.shape[0], text.shape[1], D)
